# baseline (speedup 1.0000x reference)
import functools

import jax
import jax.numpy as jnp
from jax import lax
from jax.experimental import pallas as pl
from jax.experimental.pallas import tpu as pltpu
from jax.experimental.pallas import tpu_sc as plsc

F = 26
V = 100000
D = 64
B = 4096
NC = 2
NS = 16
NW = NC * NS
DPW = D // NW
NG = B // 16
HV = V // 2

_mesh = plsc.VectorSubcoreMesh(core_axis_name="c", subcore_axis_name="s")
_params = pltpu.CompilerParams(needs_layout_passes=False)


@functools.partial(
    pl.kernel,
    mesh=_mesh,
    compiler_params=_params,
    out_type=jax.ShapeDtypeStruct((NW, B), jnp.float32),
    scratch_types=[
        pltpu.VMEM((V,), jnp.float32),
        pltpu.VMEM((B,), jnp.int32),
        pltpu.VMEM((B,), jnp.float32),
        pltpu.VMEM((B,), jnp.float32),
        pltpu.VMEM((B,), jnp.float32),
        pltpu.VMEM((B,), jnp.float32),
        pltpu.SemaphoreType.DMA,
        pltpu.SemaphoreType.DMA,
    ],
)
def _fm_part(emb_hbm, xt_hbm, lin_hbm, out_hbm,
             row_v, xidx, s0, s1, t, part, sem0, sem1):
    w = lax.axis_index("s") * NC + lax.axis_index("c")
    d0 = w * DPW

    zero = jnp.zeros((16,), jnp.float32)

    @plsc.parallel_loop(0, NG, unroll=8)
    def _(g):
        sl = pl.ds(g * 16, 16)
        s0[sl] = zero
        s1[sl] = zero
        t[sl] = zero

    def row_accum(s_ref):
        @plsc.parallel_loop(0, NG, unroll=16)
        def _(g):
            sl = pl.ds(g * 16, 16)
            e = plsc.load_gather(row_v, [xidx[sl]])
            s_ref[sl] = s_ref[sl] + e
            t[sl] = t[sl] + e * e

    def field_body(f, _):
        pltpu.sync_copy(xt_hbm.at[f], xidx)
        pltpu.sync_copy(emb_hbm.at[f * D + d0], row_v)
        row_accum(s0)
        pltpu.sync_copy(emb_hbm.at[f * D + d0 + 1], row_v)
        row_accum(s1)
        return 0

    lax.fori_loop(0, F, field_body, 0)

    @plsc.parallel_loop(0, NG, unroll=8)
    def _(g):
        sl = pl.ds(g * 16, 16)
        a, b_, c = s0[sl], s1[sl], t[sl]
        part[sl] = 0.5 * (a * a + b_ * b_ - c)

    @pl.when(w < F)
    def _():
        pltpu.sync_copy(xt_hbm.at[w], xidx)
        pltpu.sync_copy(lin_hbm.at[w], row_v)

        @plsc.parallel_loop(0, NG, unroll=8)
        def _(g):
            sl = pl.ds(g * 16, 16)
            part[sl] = part[sl] + plsc.load_gather(row_v, [xidx[sl]])

    pltpu.sync_copy(part, out_hbm.at[w])


def _combine_body(parts_ref, bias_ref, out_ref):
    acc = jnp.sum(parts_ref[...], axis=0, keepdims=True) + bias_ref[0, 0]
    out_ref[...] = 1.0 / (1.0 + jnp.exp(-acc))


_fm_combine = pl.pallas_call(
    _combine_body,
    out_shape=jax.ShapeDtypeStruct((1, B), jnp.float32),
)


def kernel(x, emb_tables, lin_tables, bias):
    emb_t = jnp.swapaxes(emb_tables, 1, 2).reshape(F * D, V)
    xt = x.T.astype(jnp.int32)
    lin2d = lin_tables.reshape(F, V)
    parts = _fm_part(emb_t, xt, lin2d)
    out = _fm_combine(parts, bias.reshape(1, 1))
    return out.reshape(B, 1)
